# initial kernel scaffold (unmeasured)
import jax
import jax.numpy as jnp
from jax import lax
from jax.experimental import pallas as pl
from jax.experimental.pallas import tpu as pltpu

N_DEV = 4


def _ring_hop(src, *, to_right, cid):

    def body(src_ref, out_ref, send_sem, recv_sem):
        d = lax.axis_index("i")
        step = 1 if to_right else N_DEV - 1
        dst = lax.rem(d + step, N_DEV)
        rdma = pltpu.make_async_remote_copy(
            src_ref=src_ref,
            dst_ref=out_ref,
            send_sem=send_sem,
            recv_sem=recv_sem,
            device_id=(dst,),
            device_id_type=pl.DeviceIdType.MESH,
        )
        rdma.start()
        rdma.wait()

    return pl.pallas_call(
        body,
        out_shape=jax.ShapeDtypeStruct(src.shape, src.dtype),
        in_specs=[pl.BlockSpec(memory_space=pltpu.ANY)],
        out_specs=pl.BlockSpec(memory_space=pltpu.ANY),
        scratch_shapes=[pltpu.SemaphoreType.DMA, pltpu.SemaphoreType.DMA],
        compiler_params=pltpu.CompilerParams(collective_id=cid),
    )(src)


def kernel(x, w_mat, scale_x, scale_w):
    d = lax.axis_index("i")

    partial = jnp.dot(
        x.astype(jnp.bfloat16),
        w_mat.astype(jnp.bfloat16),
        preferred_element_type=jnp.float32,
    )

    m_tot, n = partial.shape
    m = m_tot // N_DEV

    def chunk(i):
        return lax.dynamic_slice_in_dim(
            partial, lax.rem(i, N_DEV) * m, m, axis=0
        )

    cur = chunk(d + (N_DEV - 1))
    for s in range(N_DEV - 1):
        recv = _ring_hop(cur, to_right=True, cid=s)
        cur = recv + chunk(d + (2 * N_DEV - 2 - s))

    scale = scale_x[0] * scale_w[0]
    y = cur * scale
    own = y * (1.0 / (1.0 + jnp.exp(-jnp.clip(y, -60.0, 60.0))))

    out = jnp.zeros((m_tot, n), jnp.float32)
    out = lax.dynamic_update_slice_in_dim(out, own, d * m, axis=0)
    g = own
    for s in range(N_DEV - 1):
        g = _ring_hop(g, to_right=True, cid=N_DEV - 1 + s)
        idx = lax.rem(d + (2 * N_DEV - 1 - s), N_DEV)
        out = lax.dynamic_update_slice_in_dim(out, g, idx * m, axis=0)
    return out


# baseline (device time: 2824038 ns/iter reference)
import jax
import jax.numpy as jnp
from jax import lax
from jax.experimental import pallas as pl
from jax.experimental.pallas import tpu as pltpu

N_DEV = 4


def _ring_hop(src, *, to_right, cid):

    def body(src_ref, out_ref, send_sem, recv_sem):
        d = lax.axis_index("i")
        step = 1 if to_right else N_DEV - 1
        dst = lax.rem(d + step, N_DEV)
        rdma = pltpu.make_async_remote_copy(
            src_ref=src_ref,
            dst_ref=out_ref,
            send_sem=send_sem,
            recv_sem=recv_sem,
            device_id=(dst,),
            device_id_type=pl.DeviceIdType.MESH,
        )
        rdma.start()
        rdma.wait()

    return pl.pallas_call(
        body,
        out_shape=jax.ShapeDtypeStruct(src.shape, src.dtype),
        in_specs=[pl.BlockSpec(memory_space=pl.ANY)],
        out_specs=pl.BlockSpec(memory_space=pl.ANY),
        scratch_shapes=[pltpu.SemaphoreType.DMA, pltpu.SemaphoreType.DMA],
    )(src)


def kernel(x, w_mat, scale_x, scale_w):
    d = lax.axis_index("i")

    partial = jnp.dot(
        x.astype(jnp.bfloat16),
        w_mat.astype(jnp.bfloat16),
        preferred_element_type=jnp.float32,
    )

    m_tot, n = partial.shape
    m = m_tot // N_DEV

    def chunk(i):
        return lax.dynamic_slice_in_dim(
            partial, lax.rem(i, N_DEV) * m, m, axis=0
        )

    cur = chunk(d + (N_DEV - 1))
    for s in range(N_DEV - 1):
        recv = _ring_hop(cur, to_right=True, cid=s)
        cur = recv + chunk(d + (2 * N_DEV - 2 - s))

    scale = scale_x[0] * scale_w[0]
    y = cur * scale
    own = y * (1.0 / (1.0 + jnp.exp(-jnp.clip(y, -60.0, 60.0))))

    out = jnp.zeros((m_tot, n), jnp.float32)
    out = lax.dynamic_update_slice_in_dim(out, own, d * m, axis=0)
    g = own
    for s in range(N_DEV - 1):
        g = _ring_hop(g, to_right=True, cid=N_DEV - 1 + s)
        idx = lax.rem(d + (2 * N_DEV - 1 - s), N_DEV)
        out = lax.dynamic_update_slice_in_dim(out, g, idx * m, axis=0)
    return out


# device time: 2056571 ns/iter; 1.3732x vs baseline; 1.3732x over previous
import jax
import jax.numpy as jnp
from jax import lax
from jax.experimental import pallas as pl
from jax.experimental.pallas import tpu as pltpu

N_DEV = 4


def _ring_hop2(src_r, src_l):

    def body(sr_ref, sl_ref, or_ref, ol_ref, ss_r, rs_r, ss_l, rs_l):
        d = lax.axis_index("i")
        right = lax.rem(d + 1, N_DEV)
        left = lax.rem(d + N_DEV - 1, N_DEV)
        rdma_r = pltpu.make_async_remote_copy(
            src_ref=sr_ref, dst_ref=or_ref, send_sem=ss_r, recv_sem=rs_r,
            device_id=(right,), device_id_type=pl.DeviceIdType.MESH,
        )
        rdma_l = pltpu.make_async_remote_copy(
            src_ref=sl_ref, dst_ref=ol_ref, send_sem=ss_l, recv_sem=rs_l,
            device_id=(left,), device_id_type=pl.DeviceIdType.MESH,
        )
        rdma_r.start()
        rdma_l.start()
        rdma_r.wait()
        rdma_l.wait()

    return pl.pallas_call(
        body,
        out_shape=(
            jax.ShapeDtypeStruct(src_r.shape, src_r.dtype),
            jax.ShapeDtypeStruct(src_l.shape, src_l.dtype),
        ),
        in_specs=[
            pl.BlockSpec(memory_space=pl.ANY),
            pl.BlockSpec(memory_space=pl.ANY),
        ],
        out_specs=(
            pl.BlockSpec(memory_space=pl.ANY),
            pl.BlockSpec(memory_space=pl.ANY),
        ),
        scratch_shapes=[pltpu.SemaphoreType.DMA] * 4,
    )(src_r, src_l)


def kernel(x, w_mat, scale_x, scale_w):
    d = lax.axis_index("i")

    xb = x.astype(jnp.bfloat16)
    wb = w_mat.astype(jnp.bfloat16)
    half = wb.shape[1] // 2
    pa = jnp.dot(xb, wb[:, :half], preferred_element_type=jnp.float32)
    pb = jnp.dot(xb, wb[:, half:], preferred_element_type=jnp.float32)

    m_tot = pa.shape[0]
    m = m_tot // N_DEV

    def chunk(p, i):
        return lax.dynamic_slice_in_dim(p, lax.rem(i, N_DEV) * m, m, axis=0)

    cur_a = chunk(pa, d + (N_DEV - 1))
    cur_b = chunk(pb, d + 1)
    for s in range(N_DEV - 1):
        ra, rb = _ring_hop2(cur_a, cur_b)
        cur_a = ra + chunk(pa, d + (2 * N_DEV - 2 - s))
        cur_b = rb + chunk(pb, d + 2 + s)

    scale = scale_x[0] * scale_w[0]

    def silu(acc):
        y = acc * scale
        return y * (1.0 / (1.0 + jnp.exp(-jnp.clip(y, -60.0, 60.0))))

    own_a = silu(cur_a)
    own_b = silu(cur_b)

    out = jnp.zeros((m_tot, 2 * half), jnp.float32)
    out = lax.dynamic_update_slice(out, own_a, (d * m, 0))
    out = lax.dynamic_update_slice(out, own_b, (d * m, half))
    ga, gb = own_a, own_b
    for s in range(N_DEV - 1):
        ga, gb = _ring_hop2(ga, gb)
        ia = lax.rem(d + (2 * N_DEV - 1 - s), N_DEV)
        ib = lax.rem(d + 1 + s, N_DEV)
        out = lax.dynamic_update_slice(out, ga, (ia * m, 0))
        out = lax.dynamic_update_slice(out, gb, (ib * m, half))
    return out


# device time: 1917060 ns/iter; 1.4731x vs baseline; 1.0728x over previous
import jax
import jax.numpy as jnp
from jax import lax
from jax.experimental import pallas as pl
from jax.experimental.pallas import tpu as pltpu

N_DEV = 4


def _ring_hop2(src_r, src_l):

    def body(sr_ref, sl_ref, or_ref, ol_ref, ss_r, rs_r, ss_l, rs_l):
        d = lax.axis_index("i")
        right = lax.rem(d + 1, N_DEV)
        left = lax.rem(d + N_DEV - 1, N_DEV)
        rdma_r = pltpu.make_async_remote_copy(
            src_ref=sr_ref, dst_ref=or_ref, send_sem=ss_r, recv_sem=rs_r,
            device_id=(right,), device_id_type=pl.DeviceIdType.MESH,
        )
        rdma_l = pltpu.make_async_remote_copy(
            src_ref=sl_ref, dst_ref=ol_ref, send_sem=ss_l, recv_sem=rs_l,
            device_id=(left,), device_id_type=pl.DeviceIdType.MESH,
        )
        rdma_r.start()
        rdma_l.start()
        rdma_r.wait()
        rdma_l.wait()

    return pl.pallas_call(
        body,
        out_shape=(
            jax.ShapeDtypeStruct(src_r.shape, src_r.dtype),
            jax.ShapeDtypeStruct(src_l.shape, src_l.dtype),
        ),
        in_specs=[
            pl.BlockSpec(memory_space=pl.ANY),
            pl.BlockSpec(memory_space=pl.ANY),
        ],
        out_specs=(
            pl.BlockSpec(memory_space=pl.ANY),
            pl.BlockSpec(memory_space=pl.ANY),
        ),
        scratch_shapes=[pltpu.SemaphoreType.DMA] * 4,
    )(src_r, src_l)


def _all_gather_into_out(own_a, own_b):
    m, half = own_a.shape

    def body(oa_ref, ob_ref, out_ref, loc_sems, sa, ra, sb, rb):
        d = lax.axis_index("i")
        right = lax.rem(d + 1, N_DEV)
        left = lax.rem(d + N_DEV - 1, N_DEV)

        cp_a = pltpu.make_async_copy(
            oa_ref, out_ref.at[pl.ds(d * m, m), pl.ds(0, half)], loc_sems.at[0]
        )
        cp_b = pltpu.make_async_copy(
            ob_ref, out_ref.at[pl.ds(d * m, m), pl.ds(half, half)], loc_sems.at[1]
        )
        cp_a.start()
        cp_b.start()
        cp_a.wait()
        cp_b.wait()

        for s in range(N_DEV - 1):
            ia = lax.rem(d + N_DEV - s, N_DEV)
            ib = lax.rem(d + s, N_DEV)
            sl_a = (pl.ds(ia * m, m), pl.ds(0, half))
            sl_b = (pl.ds(ib * m, m), pl.ds(half, half))
            rdma_a = pltpu.make_async_remote_copy(
                src_ref=out_ref.at[sl_a], dst_ref=out_ref.at[sl_a],
                send_sem=sa.at[s], recv_sem=ra.at[s],
                device_id=(right,), device_id_type=pl.DeviceIdType.MESH,
            )
            rdma_b = pltpu.make_async_remote_copy(
                src_ref=out_ref.at[sl_b], dst_ref=out_ref.at[sl_b],
                send_sem=sb.at[s], recv_sem=rb.at[s],
                device_id=(left,), device_id_type=pl.DeviceIdType.MESH,
            )
            rdma_a.start()
            rdma_b.start()
            rdma_a.wait()
            rdma_b.wait()

    return pl.pallas_call(
        body,
        out_shape=jax.ShapeDtypeStruct((N_DEV * m, 2 * half), own_a.dtype),
        in_specs=[
            pl.BlockSpec(memory_space=pl.ANY),
            pl.BlockSpec(memory_space=pl.ANY),
        ],
        out_specs=pl.BlockSpec(memory_space=pl.ANY),
        scratch_shapes=[
            pltpu.SemaphoreType.DMA((2,)),
            pltpu.SemaphoreType.DMA((N_DEV - 1,)),
            pltpu.SemaphoreType.DMA((N_DEV - 1,)),
            pltpu.SemaphoreType.DMA((N_DEV - 1,)),
            pltpu.SemaphoreType.DMA((N_DEV - 1,)),
        ],
    )(own_a, own_b)


def kernel(x, w_mat, scale_x, scale_w):
    d = lax.axis_index("i")

    xb = x.astype(jnp.bfloat16)
    wb = w_mat.astype(jnp.bfloat16)
    half = wb.shape[1] // 2
    pa = jnp.dot(xb, wb[:, :half], preferred_element_type=jnp.float32)
    pb = jnp.dot(xb, wb[:, half:], preferred_element_type=jnp.float32)

    m_tot = pa.shape[0]
    m = m_tot // N_DEV

    def chunk(p, i):
        return lax.dynamic_slice_in_dim(p, lax.rem(i, N_DEV) * m, m, axis=0)

    cur_a = chunk(pa, d + (N_DEV - 1))
    cur_b = chunk(pb, d + 1)
    for s in range(N_DEV - 1):
        ra, rb = _ring_hop2(cur_a, cur_b)
        cur_a = ra + chunk(pa, d + (2 * N_DEV - 2 - s))
        cur_b = rb + chunk(pb, d + 2 + s)

    scale = scale_x[0] * scale_w[0]

    def silu(acc):
        y = acc * scale
        return y * (1.0 / (1.0 + jnp.exp(-jnp.clip(y, -60.0, 60.0))))

    own_a = silu(cur_a)
    own_b = silu(cur_b)

    return _all_gather_into_out(own_a, own_b)
